# baseline (speedup 1.0000x reference)
import jax
import jax.numpy as jnp
from jax.experimental import pallas as pl
from jax.experimental.pallas import tpu as pltpu

N = 50000
E = 800000
C_MID = 64
C_OUT = 64


def _final_matmul_kernel(agg_ref, w3t_ref, b3_ref, out_ref):
    acc = jnp.dot(agg_ref[...], w3t_ref[...], preferred_element_type=jnp.float32)
    acc = acc + b3_ref[...]
    acc = jnp.nan_to_num(acc, nan=0.0, posinf=1000000.0, neginf=-1000000.0)
    out_ref[...] = acc


def _final_matmul(agg, W3, b3):
    BLK = 1000
    n_pad = ((N + BLK - 1) // BLK) * BLK
    if n_pad != N:
        agg = jnp.pad(agg, ((0, n_pad - N), (0, 0)))
    out = pl.pallas_call(
        _final_matmul_kernel,
        grid=(n_pad // BLK,),
        in_specs=[
            pl.BlockSpec((BLK, C_MID), lambda i: (i, 0)),
            pl.BlockSpec((C_MID, C_OUT), lambda i: (0, 0)),
            pl.BlockSpec((1, C_OUT), lambda i: (0, 0)),
        ],
        out_specs=pl.BlockSpec((BLK, C_OUT), lambda i: (i, 0)),
        out_shape=jax.ShapeDtypeStruct((n_pad, C_OUT), jnp.float32),
    )(agg, W3.T, b3.reshape(1, C_OUT))
    return out[:N]


def kernel(x_in, pos_in, batch_in, in_index, out_index, W1, W2, W3, b3):
    pos_local = pos_in[in_index] - pos_in[out_index]
    M = jax.nn.celu(pos_local @ W1.T)
    M = jax.nn.celu(M @ W2.T)
    x_edge = x_in[in_index]
    deg = jax.ops.segment_sum(jnp.ones((E,), dtype=pos_in.dtype), out_index,
                              num_segments=N)
    deg = jnp.clip(deg, 1.0, None)
    x_edge = x_edge / deg[out_index][:, None]
    outer = (x_edge * M)
    agg = jax.ops.segment_sum(outer, out_index, num_segments=N)
    return _final_matmul(agg, W3, b3)


# speedup vs baseline: 9.5583x; 9.5583x over previous
import jax
import jax.numpy as jnp
from jax import lax
from jax.experimental import pallas as pl
from jax.experimental.pallas import tpu as pltpu
from jax.experimental.pallas import tpu_sc as plsc

N = 50000
E = 800000
HID = 16
C_MID = 64
C_OUT = 64

K = 64
KS = 64
NSUB = K // KS
NGROUP = K // 16
NODES_HALF = 25000
ACC_ROWS = 25088
ZCHUNKS = ACC_ROWS // K
NTILES = 16
WRC = ACC_ROWS // NTILES
WRC_LO = NODES_HALF - (NTILES - 1) * WRC
NW = 48 + HID * C_MID


def _sc_body(tbl, in_idx, out_idx, wf, bvec, agg, deg,
             acc, dacc, idxin, idxout, gin, gout, mbuf, rowbuf,
             onesbuf, zbuf, wspm, bspm, wsm, bsm,
             semi0, semi1, semg0, semg1, sems0, sems1):
    cid = lax.axis_index("c")
    sid = lax.axis_index("s")
    lanes = lax.iota(jnp.int32, 16)
    semi = (semi0, semi1)
    semg = (semg0, semg1)
    sems = (sems0, sems1)

    @pl.when(sid == 0)
    def _stage():
        pltpu.sync_copy(wf, wspm)
        pltpu.sync_copy(bvec, bspm)

    plsc.subcore_barrier()
    pltpu.sync_copy(wspm, wsm)
    pltpu.sync_copy(bspm, bsm)

    def zrow(r, _):
        for cc in range(4):
            mbuf[r, pl.ds(cc * 16, 16)] = jnp.zeros((16,), jnp.float32)
        return _

    lax.fori_loop(0, K, zrow, 0)

    def zrow8(r, _):
        rv = r * 2 + (lanes // 8)
        cv = lanes % 8
        plsc.store_scatter(zbuf, [rv, cv], jnp.zeros((16,), jnp.float32))
        plsc.store_scatter(onesbuf, [rv, cv],
                           jnp.where(cv == 0, 1.0, 0.0).astype(jnp.float32))
        return _

    lax.fori_loop(0, K // 2, zrow8, 0)

    def zacc(i, _):
        ch = sid + i * NTILES

        @pl.when(ch < ZCHUNKS)
        def _do():
            pltpu.sync_copy(mbuf, acc.at[pl.ds(ch * K, K)])
            pltpu.sync_copy(zbuf, dacc.at[pl.ds(ch * K, K)])

        return _

    lax.fori_loop(0, (ZCHUNKS + NTILES - 1) // NTILES, zacc, 0)
    plsc.subcore_barrier()

    b_edge = bsm[0]
    nch_total = E // K
    b_dn = b_edge // K
    b_up = jnp.where(b_edge % K == 0, b_dn, b_dn + 1)
    base_chunk = jnp.where(cid == 0, 0, b_dn)
    num_ch = jnp.where(cid == 0, b_up, nch_total - b_dn)
    node_base = cid * NODES_HALF
    ntrip = jnp.maximum((num_ch - sid + NTILES - 1) // NTILES, 0)

    def idx_start(b, k):
        off = (base_chunk + sid + k * NTILES) * K
        pltpu.async_copy(in_idx.at[pl.ds(off, K)], idxin.at[b], semi[b])
        pltpu.async_copy(out_idx.at[pl.ds(off, K)], idxout.at[b], semi[b])

    def idx_wait(b):
        pltpu.make_async_copy(in_idx.at[pl.ds(0, K)], idxin.at[b],
                              semi[b]).wait()
        pltpu.make_async_copy(out_idx.at[pl.ds(0, K)], idxout.at[b],
                              semi[b]).wait()

    def gather_start(b):
        for s in range(NSUB):
            sl = pl.ds(s * KS, KS)
            pltpu.async_copy(tbl.at[idxin.at[b].at[sl]], gin.at[b].at[sl],
                             semg[b])
            pltpu.async_copy(tbl.at[idxout.at[b].at[sl]], gout.at[b].at[sl],
                             semg[b])

    def gather_wait(b):
        for s in range(NSUB):
            sl = pl.ds(s * KS, KS)
            pltpu.make_async_copy(tbl.at[idxin.at[b].at[sl]],
                                  gin.at[b].at[sl], semg[b]).wait()
            pltpu.make_async_copy(tbl.at[idxout.at[b].at[sl]],
                                  gout.at[b].at[sl], semg[b]).wait()

    def scatter_start():
        for s in range(NSUB):
            sl = pl.ds(s * KS, KS)
            pltpu.async_copy(mbuf.at[sl], acc.at[rowbuf.at[s]],
                             sems0, add=True)
            pltpu.async_copy(onesbuf.at[sl], dacc.at[rowbuf.at[s]],
                             sems0, add=True)

    def scatter_wait():
        for s in range(NSUB):
            sl = pl.ds(s * KS, KS)
            pltpu.make_async_copy(mbuf.at[sl],
                                  acc.at[rowbuf.at[s]], sems0).wait()
            pltpu.make_async_copy(onesbuf.at[sl],
                                  dacc.at[rowbuf.at[s]], sems0).wait()

    def compute(b):
        gin_b = gin.at[b]
        gout_b = gout.at[b]
        mbuf_b = mbuf

        def group(g, _):
            rbase = g * 16
            ridx = rbase + lanes
            c_0 = jnp.zeros((16,), jnp.int32)
            dx = (plsc.load_gather(gin_b, [ridx, c_0])
                  - plsc.load_gather(gout_b, [ridx, c_0]))
            dy = (plsc.load_gather(gin_b, [ridx, c_0 + 1])
                  - plsc.load_gather(gout_b, [ridx, c_0 + 1]))
            dz = (plsc.load_gather(gin_b, [ridx, c_0 + 2])
                  - plsc.load_gather(gout_b, [ridx, c_0 + 2]))
            xv = plsc.load_gather(gin_b, [ridx, c_0 + 3])

            ov = idxout.at[b][pl.ds(rbase, 16)]
            local = ov - node_base
            valid = (local >= 0) & (local < NODES_HALF)
            row = jnp.where(valid, local, NODES_HALF)
            rowbuf[g // (KS // 16), pl.ds((g % (KS // 16)) * 16, 16)] = row

            hs = []
            for j in range(HID):
                h = dx * wsm[j * 3 + 0] + dy * wsm[j * 3 + 1] + dz * wsm[j * 3 + 2]
                h = jnp.where(h > 0, h, jnp.exp(h) - 1.0)
                hs.append(h)

            def cquad(cq, _):
                for u in range(4):
                    c = cq * 4 + u
                    wbase = 48 + c * HID
                    ma = hs[0] * wsm[wbase + 0]
                    mb = hs[1] * wsm[wbase + 1]
                    for j in range(2, HID, 2):
                        ma = ma + hs[j] * wsm[wbase + j]
                        mb = mb + hs[j + 1] * wsm[wbase + j + 1]
                    m = ma + mb
                    m = jnp.where(m > 0, m, jnp.exp(m) - 1.0) * xv
                    plsc.store_scatter(mbuf_b, [ridx, c_0 + c], m)
                return _

            lax.fori_loop(0, C_MID // 4, cquad, 0)
            return _

        lax.fori_loop(0, NGROUP, group, 0)

    @pl.when(ntrip > 0)
    def _pro0():
        idx_start(0, 0)
        idx_wait(0)
        gather_start(0)

    @pl.when(ntrip > 1)
    def _pro1():
        idx_start(1, 1)

    def pair_body(ip, _):
        k0 = 2 * ip
        k1 = k0 + 1
        k2 = k0 + 2
        k3 = k0 + 3

        @pl.when(k1 < ntrip)
        def _g1():
            idx_wait(1)
            gather_start(1)

        @pl.when(k0 < ntrip)
        def _c0():
            gather_wait(0)

            @pl.when(ip > 0)
            def _w0():
                scatter_wait()

            @pl.when(k2 < ntrip)
            def _i2():
                idx_start(0, k2)

            compute(0)
            scatter_start()

        @pl.when(k2 < ntrip)
        def _g2():
            idx_wait(0)
            gather_start(0)

        @pl.when(k1 < ntrip)
        def _c1():
            gather_wait(1)
            scatter_wait()

            @pl.when(k3 < ntrip)
            def _i3():
                idx_start(1, k3)

            compute(1)
            scatter_start()

        return _

    lax.fori_loop(0, (ntrip + 1) // 2, pair_body, 0)

    @pl.when(ntrip > 0)
    def _d0():
        scatter_wait()

    plsc.subcore_barrier()

    start = sid * WRC
    gbase = node_base + start
    pltpu.sync_copy(acc.at[pl.ds(start, WRC_LO)], agg.at[pl.ds(gbase, WRC_LO)])
    pltpu.sync_copy(dacc.at[pl.ds(start, WRC_LO)], deg.at[pl.ds(gbase, WRC_LO)])

    @pl.when(sid < NTILES - 1)
    def _rest():
        pltpu.sync_copy(acc.at[pl.ds(start + WRC_LO, WRC - WRC_LO)],
                        agg.at[pl.ds(gbase + WRC_LO, WRC - WRC_LO)])
        pltpu.sync_copy(dacc.at[pl.ds(start + WRC_LO, WRC - WRC_LO)],
                        deg.at[pl.ds(gbase + WRC_LO, WRC - WRC_LO)])


def _sc_call(tbl, in_idx, out_idx, wf, bvec):
    mesh = plsc.VectorSubcoreMesh(core_axis_name="c", subcore_axis_name="s")
    f = pl.kernel(
        _sc_body,
        out_type=(
            jax.ShapeDtypeStruct((N, C_MID), jnp.float32),
            jax.ShapeDtypeStruct((N, 8), jnp.float32),
        ),
        mesh=mesh,
        compiler_params=pltpu.CompilerParams(
            needs_layout_passes=False,
            use_tc_tiling_on_sc=False,
        ),
        scratch_types=(
            pltpu.VMEM_SHARED((ACC_ROWS, C_MID), jnp.float32),
            pltpu.VMEM_SHARED((ACC_ROWS, 8), jnp.float32),
            pltpu.VMEM((2, K), jnp.int32),
            pltpu.VMEM((2, K), jnp.int32),
            pltpu.VMEM((2, K, 16), jnp.float32),
            pltpu.VMEM((2, K, 16), jnp.float32),
            pltpu.VMEM((K, C_MID), jnp.float32),
            pltpu.VMEM((NSUB, KS), jnp.int32),
            pltpu.VMEM((K, 8), jnp.float32),
            pltpu.VMEM((K, 8), jnp.float32),
            pltpu.VMEM_SHARED((NW,), jnp.float32),
            pltpu.VMEM_SHARED((16,), jnp.int32),
            pltpu.SMEM((NW,), jnp.float32),
            pltpu.SMEM((16,), jnp.int32),
            pltpu.SemaphoreType.DMA,
            pltpu.SemaphoreType.DMA,
            pltpu.SemaphoreType.DMA,
            pltpu.SemaphoreType.DMA,
            pltpu.SemaphoreType.DMA,
            pltpu.SemaphoreType.DMA,
        ),
    )
    return f(tbl, in_idx, out_idx, wf, bvec)


def _final_matmul_kernel(agg_ref, deg_ref, w3t_ref, b3_ref, out_ref):
    d = jnp.maximum(deg_ref[:, 0:1], 1.0)
    a = agg_ref[...] / d
    acc = jnp.dot(a, w3t_ref[...], preferred_element_type=jnp.float32)
    acc = acc + b3_ref[...]
    out_ref[...] = jnp.nan_to_num(acc, nan=0.0, posinf=1000000.0,
                                  neginf=-1000000.0)


def _final_matmul(agg, deg, W3, b3):
    BLK = 1000
    return pl.pallas_call(
        _final_matmul_kernel,
        grid=(N // BLK,),
        in_specs=[
            pl.BlockSpec((BLK, C_MID), lambda i: (i, 0)),
            pl.BlockSpec((BLK, 8), lambda i: (i, 0)),
            pl.BlockSpec((C_MID, C_OUT), lambda i: (0, 0)),
            pl.BlockSpec((1, C_OUT), lambda i: (0, 0)),
        ],
        out_specs=pl.BlockSpec((BLK, C_OUT), lambda i: (i, 0)),
        out_shape=jax.ShapeDtypeStruct((N, C_OUT), jnp.float32),
    )(agg, deg, W3.T, b3.reshape(1, C_OUT))


def kernel(x_in, pos_in, batch_in, in_index, out_index, W1, W2, W3, b3):
    tbl = jnp.pad(jnp.concatenate([pos_in, x_in], axis=1),
                  ((0, 0), (0, 12)))
    wf = jnp.concatenate([W1.reshape(-1), W2.reshape(-1)])
    b_edge = jnp.searchsorted(out_index, NODES_HALF).astype(jnp.int32)
    bvec = jnp.full((16,), b_edge, jnp.int32)
    agg, deg = _sc_call(tbl, in_index, out_index, wf, bvec)
    return _final_matmul(agg, deg, W3, b3)
